# baseline (speedup 1.0000x reference)
import functools

import jax
import jax.numpy as jnp
import numpy as np
from jax.experimental import pallas as pl
from jax.experimental.pallas import tpu as pltpu
from jax.sharding import Mesh, PartitionSpec as P

N = 32768
V = 6890
VP = 6912
K = 8
H = 128
BASE_SCALE = 0.02
BN = 512
GRID = N // BN
DA = 36


def _prep_kernel(vx_ref, pose_ref, w1x_ref, w1p_ref, b1_ref, w2_ref, b2_ref,
                 w3_ref, b3_ref, evs_ref, rotq_ref, lbs_ref, vn_ref,
                 a_ref, ldisp_ref):
    vx = vx_ref[:, 0:3]
    pose_proj = jnp.dot(pose_ref[0:1, :], w1p_ref[:],
                        preferred_element_type=jnp.float32,
                        precision=jax.lax.Precision.HIGHEST)
    h = jnp.dot(vx, w1x_ref[0:3, :], preferred_element_type=jnp.float32,
                precision=jax.lax.Precision.HIGHEST)
    h = jnp.maximum(h + pose_proj + b1_ref[0:1, :], 0.0)
    h = jnp.dot(h, w2_ref[:], preferred_element_type=jnp.float32,
                precision=jax.lax.Precision.HIGHEST)
    h = jnp.maximum(h + b2_ref[0:1, :], 0.0)
    disp = jnp.dot(h, w3_ref[:, 0:3], preferred_element_type=jnp.float32,
                   precision=jax.lax.Precision.HIGHEST) + b3_ref[0:1, 0:3]
    rotq = rotq_ref[:]
    qn = rotq / (jnp.sqrt(jnp.sum(rotq * rotq, axis=1, keepdims=True)) + 1e-12)
    a_ref[:] = jnp.concatenate(
        [disp, jnp.exp(evs_ref[:]), qn, lbs_ref[:], vn_ref[:]], axis=1)
    rowid = jax.lax.broadcasted_iota(jnp.int32, (VP, 1), 0)
    dn = jnp.sqrt(jnp.sum(disp * disp, axis=1, keepdims=True))
    s = jnp.sum(jnp.where(rowid < V, dn, 0.0)) / V
    ldisp_ref[:] = jnp.full((8, 128), s, jnp.float32)


def _main_kernel(x_ref, yt_ref, a_ref, out_ref, part_ref):
    x0 = x_ref[:, 0:1]
    x1 = x_ref[:, 1:2]
    x2 = x_ref[:, 2:3]
    y0 = yt_ref[0:1, :]
    y1 = yt_ref[1:2, :]
    y2 = yt_ref[2:3, :]
    d0 = x0 - y0
    d1 = x1 - y1
    d2 = x2 - y2
    dsq = d0 * d0 + d1 * d1 + d2 * d2

    xb = x_ref[:, 0:3].astype(jnp.bfloat16)
    yb = yt_ref[0:3, :].astype(jnp.bfloat16)
    mm = jax.lax.dot_general(xb, yb, (((1,), (0,)), ((), ())),
                             preferred_element_type=jnp.float32)
    sx = x0 * x0 + x1 * x1 + x2 * x2
    sy = y0 * y0 + y1 * y1 + y2 * y2
    dsel = (sx + sy) - 2.0 * mm

    def _fold(arr, k):
        half = arr.shape[1] // 2
        a, b = arr[:, :half], arr[:, half:]
        return (jnp.minimum(a, b), k), (jnp.maximum(a, b), max(k // 2, 1))

    dselp = jnp.concatenate(
        [dsel, jnp.full((dsel.shape[0], 8192 - VP), jnp.inf, jnp.float32)],
        axis=1)
    pieces = [(dselp, K)]
    for _ in range(4):
        nxt = []
        for arr, k in pieces:
            nxt.extend(_fold(arr, k))
        pieces = nxt

    vals = []
    for arr, k in pieces:
        work = arr
        for i in range(k):
            m = jnp.min(work, axis=1, keepdims=True)
            vals.append(m)
            if i < k - 1:
                work = jnp.where(work <= m, jnp.inf, work)
    cand = jnp.concatenate(vals, axis=1)
    work = cand
    m = jnp.min(work, axis=1, keepdims=True)
    for _ in range(K - 1):
        work = jnp.where(work <= m, jnp.inf, work)
        m = jnp.min(work, axis=1, keepdims=True)
    thresh = m

    sel = dsel <= thresh
    w_raw = jax.lax.rsqrt(jnp.maximum(dsq, 1e-30))
    w = jnp.where(sel, w_raw, 0.0)
    wsum = jnp.sum(w, axis=1, keepdims=True)
    dsum = jnp.sum(dsq * w)

    s = jnp.dot(w.astype(jnp.bfloat16), a_ref[:].astype(jnp.bfloat16),
                preferred_element_type=jnp.float32)
    s = s * (1.0 / (wsum + 1e-7))

    new_xyz = s[:, 0:3] + x_ref[:, 0:3]
    raw = s[:, 3:5]
    new_rot = s[:, 5:9]
    out_ref[:] = jnp.concatenate(
        [new_xyz, jnp.log(raw), new_rot, s[:, 9:33]], axis=1)

    bsum = jnp.sum(jnp.maximum(raw - BASE_SCALE, 0.0))
    qn = new_rot / (jnp.sqrt(jnp.sum(new_rot * new_rot, axis=1,
                                     keepdims=True)) + 1e-9)
    qw, qx, qy, qz = qn[:, 0:1], qn[:, 1:2], qn[:, 2:3], qn[:, 3:4]
    g0 = 2.0 * (qx * qz + qw * qy)
    g1 = 2.0 * (qy * qz - qw * qx)
    g2 = 1.0 - 2.0 * (qx * qx + qy * qy)
    inorm = s[:, 33:36]
    n0 = g0 - inorm[:, 0:1]
    n1 = g1 - inorm[:, 1:2]
    n2 = g2 - inorm[:, 2:3]
    mnsum = jnp.sum(jnp.sqrt(n0 * n0 + n1 * n1 + n2 * n2))

    lane = jax.lax.broadcasted_iota(jnp.int32, (1, 1, 128), 2)
    pv = jnp.where(lane == 0, dsum,
                   jnp.where(lane == 1, bsum,
                             jnp.where(lane == 2, mnsum, 0.0)))
    part_ref[:] = pv


def _pad_rows(a, rows, value=0.0):
    return jnp.pad(a, ((0, rows - a.shape[0]), (0, 0)), constant_values=value)


def _shard_fn(gx, rots, vertex_xyz, vertex_normal, LBS_weight,
              vertex_scaling, vertex_rotation, W1, b1, W2, b2, W3, b3,
              axis=None):
    f32 = jnp.float32
    ez = jnp.arctan2(rots[:, 1, 0], rots[:, 0, 0])
    ey = jnp.arcsin(jnp.clip(-rots[:, 2, 0], -1.0, 1.0))
    ex = jnp.arctan2(rots[:, 2, 1], rots[:, 2, 2])
    pose = jnp.stack([ez, ey, ex], axis=-1).reshape(-1)

    vx_p = _pad_rows(vertex_xyz, VP)
    yt = jnp.pad(vertex_xyz.T, ((0, 5), (0, VP - V)), constant_values=1e4)
    pose8 = jnp.broadcast_to(pose[None, :], (8, 72)).astype(f32)
    w1x = jnp.pad(W1[0:3], ((0, 5), (0, 0)))
    w1p = W1[3:75]
    b1b = jnp.broadcast_to(b1[None, :], (8, H))
    b2b = jnp.broadcast_to(b2[None, :], (8, H))
    b3b = jnp.broadcast_to(b3[None, :], (8, 3))
    evs = _pad_rows(vertex_scaling, VP)
    rotq = _pad_rows(vertex_rotation, VP, 1.0)
    lbsw = _pad_rows(LBS_weight, VP)
    vn = _pad_rows(vertex_normal, VP)

    prep = pl.pallas_call(
        _prep_kernel,
        out_shape=(
            jax.ShapeDtypeStruct((VP, DA), f32),
            jax.ShapeDtypeStruct((8, 128), f32),
        ),
    )
    a_tab, ldisp_arr = prep(vx_p, pose8, w1x, w1p, b1b, W2, b2b, W3, b3b,
                            evs, rotq, lbsw, vn)

    nloc = gx.shape[0]
    grid = nloc // BN
    main = pl.pallas_call(
        _main_kernel,
        grid=(grid,),
        in_specs=[
            pl.BlockSpec((BN, 3), lambda i: (i, 0)),
            pl.BlockSpec((8, VP), lambda i: (0, 0)),
            pl.BlockSpec((VP, DA), lambda i: (0, 0)),
        ],
        out_specs=[
            pl.BlockSpec((BN, 33), lambda i: (i, 0)),
            pl.BlockSpec((1, 1, 128), lambda i: (i, 0, 0)),
        ],
        out_shape=(
            jax.ShapeDtypeStruct((nloc, 33), f32),
            jax.ShapeDtypeStruct((grid, 1, 128), f32),
        ),
        compiler_params=pltpu.CompilerParams(
            dimension_semantics=("parallel",)),
    )
    out, parts = main(gx, yt, a_tab)

    def _allsum(v):
        return jax.lax.psum(v, axis) if axis is not None else v

    dsum = _allsum(jnp.sum(parts[:, 0, 0]))
    bsum = _allsum(jnp.sum(parts[:, 0, 1]))
    msum = _allsum(jnp.sum(parts[:, 0, 2]))
    losses = jnp.stack([dsum / (N * K), ldisp_arr[0, 0],
                        bsum / (N * 2), msum / N])
    return out, losses


@functools.partial(jax.jit, static_argnums=())
def kernel(gaussians_xyz, rots, vertex_xyz, vertex_normal, LBS_weight,
           vertex_scaling, vertex_rotation, W1, b1, W2, b2, W3, b3):
    args = (gaussians_xyz, rots, vertex_xyz, vertex_normal, LBS_weight,
            vertex_scaling, vertex_rotation, W1, b1, W2, b2, W3, b3)
    devs = jax.devices()
    if len(devs) >= 2:
        mesh = Mesh(np.array(devs[:2]), ("x",))
        fn = jax.shard_map(
            functools.partial(_shard_fn, axis="x"), mesh=mesh,
            in_specs=(P("x", None),) + (P(),) * 12,
            out_specs=(P("x", None), P()),
            check_vma=False)
        return fn(*args)
    return _shard_fn(*args)


# speedup vs baseline: 1.0805x; 1.0805x over previous
import functools

import jax
import jax.numpy as jnp
import numpy as np
from jax.experimental import pallas as pl
from jax.experimental.pallas import tpu as pltpu
from jax.sharding import Mesh, PartitionSpec as P

N = 32768
V = 6890
VP = 6912
K = 8
H = 128
BASE_SCALE = 0.02
BN = 512
GRID = N // BN
DA = 36


def _prep_kernel(vx_ref, pose_ref, w1x_ref, w1p_ref, b1_ref, w2_ref, b2_ref,
                 w3_ref, b3_ref, evs_ref, rotq_ref, lbs_ref, vn_ref,
                 a_ref, ldisp_ref):
    bf = jnp.bfloat16
    vx = vx_ref[:, 0:3]
    pose_proj = jnp.dot(pose_ref[0:1, :].astype(bf), w1p_ref[:].astype(bf),
                        preferred_element_type=jnp.float32)
    h = jnp.dot(vx.astype(bf), w1x_ref[0:3, :].astype(bf),
                preferred_element_type=jnp.float32)
    h = jnp.maximum(h + pose_proj + b1_ref[0:1, :], 0.0)
    h = jnp.dot(h.astype(bf), w2_ref[:].astype(bf),
                preferred_element_type=jnp.float32)
    h = jnp.maximum(h + b2_ref[0:1, :], 0.0)
    disp = jnp.dot(h.astype(bf), w3_ref[:, 0:3].astype(bf),
                   preferred_element_type=jnp.float32) + b3_ref[0:1, 0:3]
    rotq = rotq_ref[:]
    qn = rotq / (jnp.sqrt(jnp.sum(rotq * rotq, axis=1, keepdims=True)) + 1e-12)
    a_ref[:] = jnp.concatenate(
        [disp, jnp.exp(evs_ref[:]), qn, lbs_ref[:], vn_ref[:]], axis=1)
    rowid = jax.lax.broadcasted_iota(jnp.int32, (VP, 1), 0)
    dn = jnp.sqrt(jnp.sum(disp * disp, axis=1, keepdims=True))
    s = jnp.sum(jnp.where(rowid < V, dn, 0.0)) / V
    ldisp_ref[:] = jnp.full((8, 128), s, jnp.float32)


def _main_kernel(x_ref, yt_ref, a_ref, out_ref, part_ref):
    x0 = x_ref[:, 0:1]
    x1 = x_ref[:, 1:2]
    x2 = x_ref[:, 2:3]
    y0 = yt_ref[0:1, :]
    y1 = yt_ref[1:2, :]
    y2 = yt_ref[2:3, :]
    d0 = x0 - y0
    d1 = x1 - y1
    d2 = x2 - y2
    dsq = d0 * d0 + (d1 * d1 + (d2 * d2 + 1e-30))

    xb = x_ref[:, 0:3].astype(jnp.bfloat16)
    yb = yt_ref[0:3, :].astype(jnp.bfloat16)
    mm = jax.lax.dot_general(xb, yb, (((1,), (0,)), ((), ())),
                             preferred_element_type=jnp.float32)
    sx = x0 * x0 + x1 * x1 + x2 * x2
    sy = y0 * y0 + y1 * y1 + y2 * y2
    dsel = (sx + sy) - 2.0 * mm

    def _fold(arr, k):
        half = arr.shape[1] // 2
        a, b = arr[:, :half], arr[:, half:]
        return (jnp.minimum(a, b), k), (jnp.maximum(a, b), max(k // 2, 1))

    dselp = jnp.concatenate(
        [dsel, jnp.full((dsel.shape[0], 8192 - VP), jnp.inf, jnp.float32)],
        axis=1)
    pieces = [(dselp, K)]
    for _ in range(4):
        nxt = []
        for arr, k in pieces:
            nxt.extend(_fold(arr, k))
        pieces = nxt

    vals = []
    for arr, k in pieces:
        work = arr
        for i in range(k):
            m = jnp.min(work, axis=1, keepdims=True)
            vals.append(m)
            if i < k - 1:
                work = jnp.where(work <= m, jnp.inf, work)
    cand = jnp.concatenate(vals, axis=1)
    work = cand
    tops = []
    m = jnp.min(work, axis=1, keepdims=True)
    tops.append(m)
    for _ in range(K - 1):
        work = jnp.where(work <= m, jnp.inf, work)
        m = jnp.min(work, axis=1, keepdims=True)
        tops.append(m)
    thresh = m
    dsum = jnp.sum(jnp.sqrt(jnp.maximum(jnp.concatenate(tops, axis=1), 0.0)))

    sel = dsel <= thresh
    w_raw = jax.lax.rsqrt(dsq)
    w = jnp.where(sel, w_raw, 0.0)
    wsum = jnp.sum(w, axis=1, keepdims=True)

    s = jnp.dot(w.astype(jnp.bfloat16), a_ref[:].astype(jnp.bfloat16),
                preferred_element_type=jnp.float32)
    s = s * (1.0 / (wsum + 1e-7))

    new_xyz = s[:, 0:3] + x_ref[:, 0:3]
    raw = s[:, 3:5]
    new_rot = s[:, 5:9]
    out_ref[:] = jnp.concatenate(
        [new_xyz, jnp.log(raw), new_rot, s[:, 9:33]], axis=1)

    bsum = jnp.sum(jnp.maximum(raw - BASE_SCALE, 0.0))
    qn = new_rot / (jnp.sqrt(jnp.sum(new_rot * new_rot, axis=1,
                                     keepdims=True)) + 1e-9)
    qw, qx, qy, qz = qn[:, 0:1], qn[:, 1:2], qn[:, 2:3], qn[:, 3:4]
    g0 = 2.0 * (qx * qz + qw * qy)
    g1 = 2.0 * (qy * qz - qw * qx)
    g2 = 1.0 - 2.0 * (qx * qx + qy * qy)
    inorm = s[:, 33:36]
    n0 = g0 - inorm[:, 0:1]
    n1 = g1 - inorm[:, 1:2]
    n2 = g2 - inorm[:, 2:3]
    mnsum = jnp.sum(jnp.sqrt(n0 * n0 + n1 * n1 + n2 * n2))

    lane = jax.lax.broadcasted_iota(jnp.int32, (1, 1, 128), 2)
    pv = jnp.where(lane == 0, dsum,
                   jnp.where(lane == 1, bsum,
                             jnp.where(lane == 2, mnsum, 0.0)))
    part_ref[:] = pv


def _pad_rows(a, rows, value=0.0):
    return jnp.pad(a, ((0, rows - a.shape[0]), (0, 0)), constant_values=value)


def _shard_fn(gx, rots, vertex_xyz, vertex_normal, LBS_weight,
              vertex_scaling, vertex_rotation, W1, b1, W2, b2, W3, b3,
              axis=None):
    f32 = jnp.float32
    ez = jnp.arctan2(rots[:, 1, 0], rots[:, 0, 0])
    ey = jnp.arcsin(jnp.clip(-rots[:, 2, 0], -1.0, 1.0))
    ex = jnp.arctan2(rots[:, 2, 1], rots[:, 2, 2])
    pose = jnp.stack([ez, ey, ex], axis=-1).reshape(-1)

    vx_p = _pad_rows(vertex_xyz, VP)
    yt = jnp.pad(vertex_xyz.T, ((0, 5), (0, VP - V)), constant_values=1e4)
    pose8 = jnp.broadcast_to(pose[None, :], (8, 72)).astype(f32)
    w1x = jnp.pad(W1[0:3], ((0, 5), (0, 0)))
    w1p = W1[3:75]
    b1b = jnp.broadcast_to(b1[None, :], (8, H))
    b2b = jnp.broadcast_to(b2[None, :], (8, H))
    b3b = jnp.broadcast_to(b3[None, :], (8, 3))
    evs = _pad_rows(vertex_scaling, VP)
    rotq = _pad_rows(vertex_rotation, VP, 1.0)
    lbsw = _pad_rows(LBS_weight, VP)
    vn = _pad_rows(vertex_normal, VP)

    prep = pl.pallas_call(
        _prep_kernel,
        out_shape=(
            jax.ShapeDtypeStruct((VP, DA), f32),
            jax.ShapeDtypeStruct((8, 128), f32),
        ),
    )
    a_tab, ldisp_arr = prep(vx_p, pose8, w1x, w1p, b1b, W2, b2b, W3, b3b,
                            evs, rotq, lbsw, vn)

    nloc = gx.shape[0]
    grid = nloc // BN
    main = pl.pallas_call(
        _main_kernel,
        grid=(grid,),
        in_specs=[
            pl.BlockSpec((BN, 3), lambda i: (i, 0)),
            pl.BlockSpec((8, VP), lambda i: (0, 0)),
            pl.BlockSpec((VP, DA), lambda i: (0, 0)),
        ],
        out_specs=[
            pl.BlockSpec((BN, 33), lambda i: (i, 0)),
            pl.BlockSpec((1, 1, 128), lambda i: (i, 0, 0)),
        ],
        out_shape=(
            jax.ShapeDtypeStruct((nloc, 33), f32),
            jax.ShapeDtypeStruct((grid, 1, 128), f32),
        ),
        compiler_params=pltpu.CompilerParams(
            dimension_semantics=("parallel",)),
    )
    out, parts = main(gx, yt, a_tab)

    def _allsum(v):
        return jax.lax.psum(v, axis) if axis is not None else v

    dsum = _allsum(jnp.sum(parts[:, 0, 0]))
    bsum = _allsum(jnp.sum(parts[:, 0, 1]))
    msum = _allsum(jnp.sum(parts[:, 0, 2]))
    losses = jnp.stack([dsum / (N * K), ldisp_arr[0, 0],
                        bsum / (N * 2), msum / N])
    return out, losses


@functools.partial(jax.jit, static_argnums=())
def kernel(gaussians_xyz, rots, vertex_xyz, vertex_normal, LBS_weight,
           vertex_scaling, vertex_rotation, W1, b1, W2, b2, W3, b3):
    args = (gaussians_xyz, rots, vertex_xyz, vertex_normal, LBS_weight,
            vertex_scaling, vertex_rotation, W1, b1, W2, b2, W3, b3)
    devs = jax.devices()
    if len(devs) >= 2:
        mesh = Mesh(np.array(devs[:2]), ("x",))
        fn = jax.shard_map(
            functools.partial(_shard_fn, axis="x"), mesh=mesh,
            in_specs=(P("x", None),) + (P(),) * 12,
            out_specs=(P("x", None), P()),
            check_vma=False)
        return fn(*args)
    return _shard_fn(*args)
